# Initial kernel scaffold; baseline (speedup 1.0000x reference)
import jax
import jax.numpy as jnp
from jax.experimental import pallas as pl


def kernel(x, w_exp, bn0_s, bn0_b, w_dw, bn1_s, bn1_b, w_sr, b_sr, w_se, b_se, w_pr, bn2_s, bn2_b):
    raise NotImplementedError("write your pallas kernel here")


# speedup vs baseline: 1.1076x; 1.1076x over previous
import functools

import jax
import jax.numpy as jnp
from jax.experimental import pallas as pl
from jax.experimental.pallas import tpu as pltpu


def _swish(x):
    return x * jax.nn.sigmoid(x)


def _mbconv_body(x_ref, w_exp_ref, w_pr_ref, slab_ref, b_sr_ref, o_ref,
                 *, K: int, H: int, W: int, residual: bool):
    HW = H * W
    Cexp = slab_ref.shape[0]
    Cout = w_pr_ref.shape[0]
    Csq = b_sr_ref.shape[1]
    K2 = K * K
    pad = (K - 1) // 2

    w_exp = w_exp_ref[...]
    w_pr = w_pr_ref[...]
    slab = slab_ref[...]
    b_sr = b_sr_ref[...]

    taps = [slab[:, i:i + 1] for i in range(K2)]
    bn0_b = slab[:, K2:K2 + 1]
    bn1_b = slab[:, K2 + 1:K2 + 2]
    b_se = slab[:, K2 + 2:K2 + 3]
    bn2_b = slab[:Cout, K2 + 3:K2 + 4]
    w_sr = slab[:, K2 + 4:K2 + 4 + Csq]
    w_se = slab[:, K2 + 4 + Csq:K2 + 4 + 2 * Csq]

    lane = jax.lax.broadcasted_iota(jnp.int32, (1, HW), 1)
    h_idx = lane // W
    w_idx = lane % W

    def sh(v, s):
        return pltpu.roll(v, shift=(-s) % HW, axis=1)

    x_b = x_ref[0]

    e = jnp.dot(w_exp, x_b.astype(jnp.bfloat16),
                preferred_element_type=jnp.float32)
    e = _swish(e + bn0_b)

    cols = {0: e}
    for ow in range(1, pad + 1):
        cols[-ow] = jnp.where(w_idx >= ow, sh(e, -ow), 0.0)
        cols[ow] = jnp.where(w_idx < W - ow, sh(e, ow), 0.0)

    acc = None
    for dh in range(K):
        oh = dh - pad
        t = None
        for dj in range(K):
            term = cols[dj - pad] * taps[dh * K + dj]
            t = term if t is None else t + term
        if oh != 0:
            m = (h_idx >= -oh) if oh < 0 else (h_idx < H - oh)
            t = jnp.where(m, sh(t, oh * W), 0.0)
        acc = t if acc is None else acc + t

    d = _swish(acc + bn1_b)

    pooled = jnp.sum(d, axis=1, keepdims=True)
    red = jnp.sum(w_sr * pooled, axis=0, keepdims=True) + b_sr
    red = _swish(red)
    ex = jnp.sum(w_se * red, axis=1, keepdims=True) + b_se
    gate = jax.nn.sigmoid(ex)
    dg = d * gate

    p = jnp.dot(w_pr, dg.astype(jnp.bfloat16),
                preferred_element_type=jnp.float32) + bn2_b

    if residual:
        p = p + x_b
    o_ref[0] = p


def _mbconv_forward(x_nchw, params, *, ksize, stride):
    B, Cin, H, W = x_nchw.shape
    HW = H * W
    x = x_nchw.reshape(B, Cin, HW).astype(jnp.float32)

    K = ksize
    K2 = K * K
    Cexp = params["w_exp"].shape[0]
    Cout = params["w_pr"].shape[0]
    Csq = params["w_sr"].shape[1]
    residual = (stride == 1 and Cin == Cout)

    w_exp_f = (params["w_exp"] * params["bn0_s"]).astype(jnp.bfloat16)
    w_pr_f = (params["w_pr"] * params["bn2_s"]).astype(jnp.bfloat16)

    taps = (params["w_dw"].reshape(K2, Cexp)
            * params["bn1_s"].reshape(1, Cexp)).T
    bn2_b_pad = jnp.zeros((Cexp, 1), jnp.float32).at[:Cout].set(params["bn2_b"])
    slab = jnp.concatenate(
        [taps,
         params["bn0_b"],
         params["bn1_b"],
         params["b_se"],
         bn2_b_pad,
         params["w_sr"] / HW,
         params["w_se"]],
        axis=1).astype(jnp.float32)
    ncols = K2 + 4 + 2 * Csq
    b_sr = params["b_sr"].astype(jnp.float32)

    in_specs = [
        pl.BlockSpec((1, Cin, HW), lambda b: (b, 0, 0)),
        pl.BlockSpec((Cexp, Cin), lambda b: (0, 0)),
        pl.BlockSpec((Cout, Cexp), lambda b: (0, 0)),
        pl.BlockSpec((Cexp, ncols), lambda b: (0, 0)),
        pl.BlockSpec((1, Csq), lambda b: (0, 0)),
    ]
    out_spec = pl.BlockSpec((1, Cout, HW), lambda b: (b, 0, 0))

    body = functools.partial(_mbconv_body, K=K, H=H, W=W, residual=residual)

    out = pl.pallas_call(
        body,
        out_shape=jax.ShapeDtypeStruct((B, Cout, HW), jnp.float32),
        grid=(B,),
        in_specs=in_specs,
        out_specs=out_spec,
        compiler_params=pltpu.CompilerParams(
            dimension_semantics=("parallel",),
            vmem_limit_bytes=64 * 1024 * 1024,
        ),
    )(x, w_exp_f, w_pr_f, slab, b_sr)

    return out.reshape(B, Cout, H, W)


def kernel(x, w_exp, bn0_s, bn0_b, w_dw, bn1_s, bn1_b, w_sr, b_sr, w_se,
           b_se, w_pr, bn2_s, bn2_b):
    params = {"w_exp": w_exp, "bn0_s": bn0_s, "bn0_b": bn0_b, "w_dw": w_dw,
              "bn1_s": bn1_s, "bn1_b": bn1_b, "w_sr": w_sr, "b_sr": b_sr,
              "w_se": w_se, "b_se": b_se, "w_pr": w_pr, "bn2_s": bn2_s,
              "bn2_b": bn2_b}
    K = int(round(w_dw.shape[0] ** 0.5))
    return _mbconv_forward(x, params, ksize=K, stride=1)


# speedup vs baseline: 1.1168x; 1.0083x over previous
import functools

import jax
import jax.numpy as jnp
from jax.experimental import pallas as pl
from jax.experimental.pallas import tpu as pltpu


def _swish(x):
    return x * jax.nn.sigmoid(x)


def _mbconv_body(x_ref, w_exp_ref, w_pr_ref, slab_ref, b_sr_ref, o_ref,
                 *, K: int, H: int, W: int, residual: bool):
    HW = H * W
    Cexp = slab_ref.shape[0]
    Cout = w_pr_ref.shape[0]
    Csq = b_sr_ref.shape[1]
    K2 = K * K
    pad = (K - 1) // 2

    w_exp = w_exp_ref[...]
    w_pr = w_pr_ref[...]
    slab = slab_ref[...]
    b_sr = b_sr_ref[...]

    taps = [slab[:, i:i + 1] for i in range(K2)]
    bn0_b = slab[:, K2:K2 + 1]
    bn1_b = slab[:, K2 + 1:K2 + 2]
    b_se = slab[:, K2 + 2:K2 + 3]
    bn2_b = slab[:Cout, K2 + 3:K2 + 4]
    w_sr = slab[:, K2 + 4:K2 + 4 + Csq]
    w_se = slab[:, K2 + 4 + Csq:K2 + 4 + 2 * Csq]

    lane = jax.lax.broadcasted_iota(jnp.int32, (1, HW), 1)
    h_idx = lane // W
    w_idx = lane % W

    def sh(v, s):
        return pltpu.roll(v, shift=(-s) % HW, axis=1)

    x_b = x_ref[0]

    e = jnp.dot(w_exp, x_b.astype(jnp.bfloat16),
                preferred_element_type=jnp.float32)
    e = _swish(e + bn0_b)

    cols = {0: e}
    for ow in range(1, pad + 1):
        m_neg = (w_idx >= ow).astype(jnp.float32)
        m_pos = (w_idx < W - ow).astype(jnp.float32)
        cols[-ow] = sh(e, -ow) * m_neg
        cols[ow] = sh(e, ow) * m_pos

    acc = None
    for dh in range(K):
        oh = dh - pad
        t = None
        for dj in range(K):
            term = cols[dj - pad] * taps[dh * K + dj]
            t = term if t is None else t + term
        if oh != 0:
            m = ((h_idx >= -oh) if oh < 0 else (h_idx < H - oh)).astype(jnp.float32)
            t = sh(t, oh * W) * m
        acc = t if acc is None else acc + t

    d = _swish(acc + bn1_b)

    pooled = jnp.sum(d, axis=1, keepdims=True)
    red = jnp.sum(w_sr * pooled, axis=0, keepdims=True) + b_sr
    red = _swish(red)
    ex = jnp.sum(w_se * red, axis=1, keepdims=True) + b_se
    gate = jax.nn.sigmoid(ex)

    w_pr_g = w_pr * gate.T.astype(jnp.bfloat16)
    p = jnp.dot(w_pr_g, d.astype(jnp.bfloat16),
                preferred_element_type=jnp.float32) + bn2_b

    if residual:
        p = p + x_b
    o_ref[0] = p


def _mbconv_forward(x_nchw, params, *, ksize, stride):
    B, Cin, H, W = x_nchw.shape
    HW = H * W
    x = x_nchw.reshape(B, Cin, HW).astype(jnp.float32)

    K = ksize
    K2 = K * K
    Cexp = params["w_exp"].shape[0]
    Cout = params["w_pr"].shape[0]
    Csq = params["w_sr"].shape[1]
    residual = (stride == 1 and Cin == Cout)

    w_exp_f = (params["w_exp"] * params["bn0_s"]).astype(jnp.bfloat16)
    w_pr_f = (params["w_pr"] * params["bn2_s"]).astype(jnp.bfloat16)

    taps = (params["w_dw"].reshape(K2, Cexp)
            * params["bn1_s"].reshape(1, Cexp)).T
    bn2_b_pad = jnp.zeros((Cexp, 1), jnp.float32).at[:Cout].set(params["bn2_b"])
    slab = jnp.concatenate(
        [taps,
         params["bn0_b"],
         params["bn1_b"],
         params["b_se"],
         bn2_b_pad,
         params["w_sr"] / HW,
         params["w_se"]],
        axis=1).astype(jnp.float32)
    ncols = K2 + 4 + 2 * Csq
    b_sr = params["b_sr"].astype(jnp.float32)

    in_specs = [
        pl.BlockSpec((1, Cin, HW), lambda b: (b, 0, 0)),
        pl.BlockSpec((Cexp, Cin), lambda b: (0, 0)),
        pl.BlockSpec((Cout, Cexp), lambda b: (0, 0)),
        pl.BlockSpec((Cexp, ncols), lambda b: (0, 0)),
        pl.BlockSpec((1, Csq), lambda b: (0, 0)),
    ]
    out_spec = pl.BlockSpec((1, Cout, HW), lambda b: (b, 0, 0))

    body = functools.partial(_mbconv_body, K=K, H=H, W=W, residual=residual)

    out = pl.pallas_call(
        body,
        out_shape=jax.ShapeDtypeStruct((B, Cout, HW), jnp.float32),
        grid=(B,),
        in_specs=in_specs,
        out_specs=out_spec,
        compiler_params=pltpu.CompilerParams(
            dimension_semantics=("parallel",),
            vmem_limit_bytes=64 * 1024 * 1024,
        ),
    )(x, w_exp_f, w_pr_f, slab, b_sr)

    return out.reshape(B, Cout, H, W)


def kernel(x, w_exp, bn0_s, bn0_b, w_dw, bn1_s, bn1_b, w_sr, b_sr, w_se,
           b_se, w_pr, bn2_s, bn2_b):
    params = {"w_exp": w_exp, "bn0_s": bn0_s, "bn0_b": bn0_b, "w_dw": w_dw,
              "bn1_s": bn1_s, "bn1_b": bn1_b, "w_sr": w_sr, "b_sr": b_sr,
              "w_se": w_se, "b_se": b_se, "w_pr": w_pr, "bn2_s": bn2_s,
              "bn2_b": bn2_b}
    K = int(round(w_dw.shape[0] ** 0.5))
    return _mbconv_forward(x, params, ksize=K, stride=1)


# speedup vs baseline: 1.5062x; 1.3487x over previous
import functools

import jax
import jax.numpy as jnp
from jax.experimental import pallas as pl
from jax.experimental.pallas import tpu as pltpu


def _swish(x):
    h = 0.5 * x
    return h + h * jnp.tanh(h)


def _sigmoid(x):
    return 0.5 + 0.5 * jnp.tanh(0.5 * x)


def _mbconv_body(x_ref, w_exp_ref, w_pr_ref, slab_ref, b_sr_ref, bn2_b_ref,
                 o_ref, t0_ref, t2_ref, ea_ref, eb_ref,
                 *, K: int, H: int, W: int, residual: bool):
    HW = H * W
    Cexp = slab_ref.shape[1]
    Cout = w_pr_ref.shape[0]
    Csq = b_sr_ref.shape[0]
    K2 = K * K
    pad = (K - 1) // 2
    assert K == 3, "3x3 depthwise path"

    w_exp = w_exp_ref[...]
    w_pr = w_pr_ref[...]
    slab = slab_ref[...]
    b_sr = b_sr_ref[...]
    bn2_b = bn2_b_ref[...]

    taps = [slab[i:i + 1, :].astype(jnp.bfloat16) for i in range(K2)]
    bn0_b = slab[K2:K2 + 1, :]
    bn1_b = slab[K2 + 1:K2 + 2, :]
    b_se = slab[K2 + 2:K2 + 3, :]
    w_sr_t = slab[K2 + 3:K2 + 3 + Csq, :]
    w_se_t = slab[K2 + 3 + Csq:K2 + 3 + 2 * Csq, :]

    r_idx = jax.lax.broadcasted_iota(jnp.int32, (HW, 1), 0)
    w_of_r = jax.lax.rem(r_idx, W)
    m_m1 = (w_of_r >= 1).astype(jnp.float32)
    m_p1 = (w_of_r < W - 1).astype(jnp.float32)

    x_b = x_ref[0]

    e_t = jax.lax.dot_general(
        x_b.astype(jnp.bfloat16), w_exp,
        (((0,), (1,)), ((), ())), preferred_element_type=jnp.float32)
    es = _swish(e_t + bn0_b)

    half = Cexp // 2
    zpad = jnp.zeros((1, half), jnp.float32)
    ea_ref[0:1, :] = zpad
    eb_ref[0:1, :] = zpad
    ea_ref[HW + 1:HW + 2, :] = zpad
    eb_ref[HW + 1:HW + 2, :] = zpad
    ea_ref[1:HW + 1, :] = es[:, 0:half]
    eb_ref[1:HW + 1, :] = es[:, half:Cexp]

    sh_m1 = jnp.concatenate([ea_ref[0:HW, :], eb_ref[0:HW, :]], axis=1)
    sh_p1 = jnp.concatenate([ea_ref[2:HW + 2, :], eb_ref[2:HW + 2, :]], axis=1)
    c_m1 = (sh_m1 * m_m1).astype(jnp.bfloat16)
    c_p1 = (sh_p1 * m_p1).astype(jnp.bfloat16)
    c_0 = es.astype(jnp.bfloat16)

    def trow(dh):
        return (taps[dh * K] * c_m1 + taps[dh * K + 1] * c_0
                + taps[dh * K + 2] * c_p1)

    t0_ref[0:W, :] = jnp.zeros((W, Cexp), jnp.bfloat16)
    t0_ref[W:W + HW, :] = trow(0)
    t2_ref[W:W + HW, :] = trow(2)
    t2_ref[W + HW:2 * W + HW, :] = jnp.zeros((W, Cexp), jnp.bfloat16)

    acc = trow(1) + t0_ref[0:HW, :] + t2_ref[2 * W:2 * W + HW, :]
    d = _swish(acc.astype(jnp.float32) + bn1_b)

    pooled = jnp.sum(d, axis=0, keepdims=True)
    red = jnp.sum(w_sr_t * pooled, axis=1, keepdims=True) + b_sr
    red = _swish(red)
    ex = jnp.sum(w_se_t * red, axis=0, keepdims=True) + b_se
    gate = _sigmoid(ex)

    w_pr_g = w_pr * gate.astype(jnp.bfloat16)
    p = jax.lax.dot_general(
        w_pr_g, d.astype(jnp.bfloat16),
        (((1,), (1,)), ((), ())), preferred_element_type=jnp.float32)
    p = p + bn2_b

    if residual:
        p = p + x_b
    o_ref[0] = p


def _mbconv_forward(x_nchw, params, *, ksize, stride):
    B, Cin, H, W = x_nchw.shape
    HW = H * W
    x = x_nchw.reshape(B, Cin, HW).astype(jnp.float32)

    K = ksize
    K2 = K * K
    Cexp = params["w_exp"].shape[0]
    Cout = params["w_pr"].shape[0]
    Csq = params["w_sr"].shape[1]
    residual = (stride == 1 and Cin == Cout)

    w_exp_f = (params["w_exp"] * params["bn0_s"]).astype(jnp.bfloat16)
    w_pr_f = (params["w_pr"] * params["bn2_s"]).astype(jnp.bfloat16)

    taps = params["w_dw"].reshape(K2, Cexp) * params["bn1_s"].reshape(1, Cexp)
    slab = jnp.concatenate(
        [taps,
         params["bn0_b"].reshape(1, Cexp),
         params["bn1_b"].reshape(1, Cexp),
         params["b_se"].reshape(1, Cexp),
         (params["w_sr"] / HW).T,
         params["w_se"].T],
        axis=0).astype(jnp.float32)
    nrows = K2 + 3 + 2 * Csq
    b_sr_c = params["b_sr"].reshape(Csq, 1).astype(jnp.float32)
    bn2_b = params["bn2_b"].reshape(Cout, 1).astype(jnp.float32)

    in_specs = [
        pl.BlockSpec((1, Cin, HW), lambda b: (b, 0, 0)),
        pl.BlockSpec((Cexp, Cin), lambda b: (0, 0)),
        pl.BlockSpec((Cout, Cexp), lambda b: (0, 0)),
        pl.BlockSpec((nrows, Cexp), lambda b: (0, 0)),
        pl.BlockSpec((Csq, 1), lambda b: (0, 0)),
        pl.BlockSpec((Cout, 1), lambda b: (0, 0)),
    ]
    out_spec = pl.BlockSpec((1, Cout, HW), lambda b: (b, 0, 0))

    body = functools.partial(_mbconv_body, K=K, H=H, W=W, residual=residual)

    out = pl.pallas_call(
        body,
        out_shape=jax.ShapeDtypeStruct((B, Cout, HW), jnp.float32),
        grid=(B,),
        in_specs=in_specs,
        out_specs=out_spec,
        scratch_shapes=[
            pltpu.VMEM((HW + 2 * W, Cexp), jnp.bfloat16),
            pltpu.VMEM((HW + 2 * W, Cexp), jnp.bfloat16),
            pltpu.VMEM((HW + 2, Cexp // 2), jnp.float32),
            pltpu.VMEM((HW + 2, Cexp // 2), jnp.float32),
        ],
        compiler_params=pltpu.CompilerParams(
            dimension_semantics=("parallel",),
            vmem_limit_bytes=64 * 1024 * 1024,
        ),
    )(x, w_exp_f, w_pr_f, slab, b_sr_c, bn2_b)

    return out.reshape(B, Cout, H, W)


def kernel(x, w_exp, bn0_s, bn0_b, w_dw, bn1_s, bn1_b, w_sr, b_sr, w_se,
           b_se, w_pr, bn2_s, bn2_b):
    params = {"w_exp": w_exp, "bn0_s": bn0_s, "bn0_b": bn0_b, "w_dw": w_dw,
              "bn1_s": bn1_s, "bn1_b": bn1_b, "w_sr": w_sr, "b_sr": b_sr,
              "w_se": w_se, "b_se": b_se, "w_pr": w_pr, "bn2_s": bn2_s,
              "bn2_b": bn2_b}
    K = int(round(w_dw.shape[0] ** 0.5))
    return _mbconv_forward(x, params, ksize=K, stride=1)


# speedup vs baseline: 1.6069x; 1.0669x over previous
import functools

import jax
import jax.numpy as jnp
from jax.experimental import pallas as pl
from jax.experimental.pallas import tpu as pltpu


def _swish(x):
    h = 0.5 * x
    return h + h * jnp.tanh(h)


def _sigmoid(x):
    return 0.5 + 0.5 * jnp.tanh(0.5 * x)


def _mbconv_body(x_ref, w_exp_ref, w_pr_ref, slab_ref, b_sr_ref, bn2_b_ref,
                 o_ref, t0_ref, t2_ref, ea_ref, eb_ref,
                 *, K: int, H: int, W: int, residual: bool, img_per_step: int):
    HW = H * W
    Cexp = slab_ref.shape[1]
    Csq = b_sr_ref.shape[0]
    K2 = K * K
    assert K == 3, "3x3 depthwise path"

    w_exp = w_exp_ref[...]
    w_pr = w_pr_ref[...]
    slab = slab_ref[...]
    b_sr = b_sr_ref[...]
    bn2_b = bn2_b_ref[...]

    taps = [slab[i:i + 1, :].astype(jnp.bfloat16) for i in range(K2)]
    bn0_b = slab[K2:K2 + 1, :]
    bn1_b = slab[K2 + 1:K2 + 2, :]
    b_se = slab[K2 + 2:K2 + 3, :]
    w_sr_t = slab[K2 + 3:K2 + 3 + Csq, :]
    w_se_t = slab[K2 + 3 + Csq:K2 + 3 + 2 * Csq, :]

    r_idx = jax.lax.broadcasted_iota(jnp.int32, (HW, 1), 0)
    w_of_r = jax.lax.rem(r_idx, W)
    m_m1 = (w_of_r >= 1).astype(jnp.float32)
    m_p1 = (w_of_r < W - 1).astype(jnp.float32)

    half = Cexp // 2
    zpad = jnp.zeros((1, half), jnp.float32)

    def one_image(i):
        x_b = x_ref[i]
        t0 = t0_ref.at[i]
        t2 = t2_ref.at[i]
        ea = ea_ref.at[i]
        eb = eb_ref.at[i]

        e_t = jax.lax.dot_general(
            x_b.astype(jnp.bfloat16), w_exp,
            (((0,), (1,)), ((), ())), preferred_element_type=jnp.float32)
        es = _swish(e_t + bn0_b)

        ea[0:1, :] = zpad
        eb[0:1, :] = zpad
        ea[HW + 1:HW + 2, :] = zpad
        eb[HW + 1:HW + 2, :] = zpad
        ea[1:HW + 1, :] = es[:, 0:half]
        eb[1:HW + 1, :] = es[:, half:Cexp]

        sh_m1 = jnp.concatenate([ea[0:HW, :], eb[0:HW, :]], axis=1)
        sh_p1 = jnp.concatenate([ea[2:HW + 2, :], eb[2:HW + 2, :]], axis=1)
        c_m1 = (sh_m1 * m_m1).astype(jnp.bfloat16)
        c_p1 = (sh_p1 * m_p1).astype(jnp.bfloat16)
        c_0 = es.astype(jnp.bfloat16)

        def trow(dh):
            return (taps[dh * K] * c_m1 + taps[dh * K + 1] * c_0
                    + taps[dh * K + 2] * c_p1)

        t0[0:W, :] = jnp.zeros((W, Cexp), jnp.bfloat16)
        t0[W:W + HW, :] = trow(0)
        t2[W:W + HW, :] = trow(2)
        t2[W + HW:2 * W + HW, :] = jnp.zeros((W, Cexp), jnp.bfloat16)

        acc = trow(1) + t0[0:HW, :] + t2[2 * W:2 * W + HW, :]
        d = _swish(acc.astype(jnp.float32) + bn1_b)

        pooled = jnp.sum(d, axis=0, keepdims=True)
        red = jnp.sum(w_sr_t * pooled, axis=1, keepdims=True) + b_sr
        red = _swish(red)
        ex = jnp.sum(w_se_t * red, axis=0, keepdims=True) + b_se
        gate = _sigmoid(ex)

        w_pr_g = w_pr * gate.astype(jnp.bfloat16)
        p = jax.lax.dot_general(
            w_pr_g, d.astype(jnp.bfloat16),
            (((1,), (1,)), ((), ())), preferred_element_type=jnp.float32)
        p = p + bn2_b

        if residual:
            p = p + x_b
        o_ref[i] = p

    for i in range(img_per_step):
        one_image(i)


def _mbconv_forward(x_nchw, params, *, ksize, stride):
    B, Cin, H, W = x_nchw.shape
    HW = H * W
    x = x_nchw.reshape(B, Cin, HW).astype(jnp.float32)

    K = ksize
    K2 = K * K
    Cexp = params["w_exp"].shape[0]
    Cout = params["w_pr"].shape[0]
    Csq = params["w_sr"].shape[1]
    residual = (stride == 1 and Cin == Cout)
    ips = 2 if B % 2 == 0 else 1

    w_exp_f = (params["w_exp"] * params["bn0_s"]).astype(jnp.bfloat16)
    w_pr_f = (params["w_pr"] * params["bn2_s"]).astype(jnp.bfloat16)

    taps = params["w_dw"].reshape(K2, Cexp) * params["bn1_s"].reshape(1, Cexp)
    slab = jnp.concatenate(
        [taps,
         params["bn0_b"].reshape(1, Cexp),
         params["bn1_b"].reshape(1, Cexp),
         params["b_se"].reshape(1, Cexp),
         (params["w_sr"] / HW).T,
         params["w_se"].T],
        axis=0).astype(jnp.float32)
    nrows = K2 + 3 + 2 * Csq
    b_sr_c = params["b_sr"].reshape(Csq, 1).astype(jnp.float32)
    bn2_b = params["bn2_b"].reshape(Cout, 1).astype(jnp.float32)

    in_specs = [
        pl.BlockSpec((ips, Cin, HW), lambda b: (b, 0, 0)),
        pl.BlockSpec((Cexp, Cin), lambda b: (0, 0)),
        pl.BlockSpec((Cout, Cexp), lambda b: (0, 0)),
        pl.BlockSpec((nrows, Cexp), lambda b: (0, 0)),
        pl.BlockSpec((Csq, 1), lambda b: (0, 0)),
        pl.BlockSpec((Cout, 1), lambda b: (0, 0)),
    ]
    out_spec = pl.BlockSpec((ips, Cout, HW), lambda b: (b, 0, 0))

    body = functools.partial(_mbconv_body, K=K, H=H, W=W, residual=residual,
                             img_per_step=ips)

    out = pl.pallas_call(
        body,
        out_shape=jax.ShapeDtypeStruct((B, Cout, HW), jnp.float32),
        grid=(B // ips,),
        in_specs=in_specs,
        out_specs=out_spec,
        scratch_shapes=[
            pltpu.VMEM((ips, HW + 2 * W, Cexp), jnp.bfloat16),
            pltpu.VMEM((ips, HW + 2 * W, Cexp), jnp.bfloat16),
            pltpu.VMEM((ips, HW + 2, Cexp // 2), jnp.float32),
            pltpu.VMEM((ips, HW + 2, Cexp // 2), jnp.float32),
        ],
        compiler_params=pltpu.CompilerParams(
            dimension_semantics=("parallel",),
            vmem_limit_bytes=60000 * 1024,
        ),
    )(x, w_exp_f, w_pr_f, slab, b_sr_c, bn2_b)

    return out.reshape(B, Cout, H, W)


def kernel(x, w_exp, bn0_s, bn0_b, w_dw, bn1_s, bn1_b, w_sr, b_sr, w_se,
           b_se, w_pr, bn2_s, bn2_b):
    params = {"w_exp": w_exp, "bn0_s": bn0_s, "bn0_b": bn0_b, "w_dw": w_dw,
              "bn1_s": bn1_s, "bn1_b": bn1_b, "w_sr": w_sr, "b_sr": b_sr,
              "w_se": w_se, "b_se": b_se, "w_pr": w_pr, "bn2_s": bn2_s,
              "bn2_b": bn2_b}
    K = int(round(w_dw.shape[0] ** 0.5))
    return _mbconv_forward(x, params, ksize=K, stride=1)
